# Initial kernel scaffold; baseline (speedup 1.0000x reference)
import jax
import jax.numpy as jnp
from jax.experimental import pallas as pl


def kernel(x):
    raise NotImplementedError("write your pallas kernel here")


# speedup vs baseline: 7.2736x; 7.2736x over previous
import jax
import jax.numpy as jnp
from jax.experimental import pallas as pl

_MIN32 = -(2**31)
_MAX32 = 2**31 - 1
_POSMASK = 0x7FFFFFFF
_K = 2047


def _median_kernel(xi_ref, o_ref):
    b = xi_ref[0]
    key = jnp.where(b < 0, b ^ _POSMASK, b)

    def body(i, p):
        t = p | jnp.left_shift(jnp.int32(1), 31 - i)
        thr = t ^ _MIN32
        cnt = jnp.sum((key < thr).astype(jnp.int32), axis=0, keepdims=True)
        return jnp.where(cnt <= _K, t, p)

    p0 = jnp.zeros((1, 128), jnp.int32)
    p = jax.lax.fori_loop(0, 32, body, p0)
    key_a = p ^ _MIN32

    le = jnp.sum((key <= key_a).astype(jnp.int32), axis=0, keepdims=True)
    gt = jnp.where(key > key_a, key, _MAX32)
    mn = jnp.min(gt, axis=0, keepdims=True)
    key_b = jnp.where(le > _K + 1, key_a, mn)

    bits_a = jnp.where(key_a >= 0, key_a, key_a ^ _POSMASK)
    bits_b = jnp.where(key_b >= 0, key_b, key_b ^ _POSMASK)
    va = jax.lax.bitcast_convert_type(bits_a, jnp.float32)
    vb = jax.lax.bitcast_convert_type(bits_b, jnp.float32)
    o_ref[0] = va + 0.5 * (vb - va)


def kernel(x):
    xi = jax.lax.bitcast_convert_type(x, jnp.int32)
    out = pl.pallas_call(
        _median_kernel,
        grid=(64,),
        in_specs=[pl.BlockSpec((1, 4096, 128), lambda i: (i, 0, 0))],
        out_specs=pl.BlockSpec((1, 1, 128), lambda i: (i, 0, 0)),
        out_shape=jax.ShapeDtypeStruct((64, 1, 128), jnp.float32),
    )(xi)
    return out.reshape(64, 128)
